# baseline (speedup 1.0000x reference)
import functools

import jax
import jax.numpy as jnp
from jax import lax
from jax.experimental import pallas as pl
from jax.experimental.pallas import tpu as pltpu
from jax.experimental.pallas import tpu_sc as plsc

NC, NS, L = 2, 16, 16
NW = NC * NS
N = 131072
C = 64
D, W, H = 5, 400, 352
NCELLS = D * W * H
P = N // NW
CW = NCELLS // NW
SENT = 1 << 30

ZB = (-3.0, -2.2, -1.4000000000000004, -0.5999999999999996,
      0.20000000000000018, 1.0000000000000002)

_mesh = plsc.VectorSubcoreMesh(core_axis_name="c", subcore_axis_name="s")


def _wid():
    return lax.axis_index("s") * NC + lax.axis_index("c")


@functools.partial(
    pl.kernel, mesh=_mesh,
    out_type=jax.ShapeDtypeStruct((N,), jnp.int32),
    scratch_types=[
        pltpu.VMEM((P,), jnp.float32),
        pltpu.VMEM((P,), jnp.float32),
        pltpu.VMEM((P,), jnp.float32),
        pltpu.VMEM((P,), jnp.int32),
    ],
)
def _cells_k(x_hbm, y_hbm, z_hbm, cells_hbm, xv, yv, zv, cv):
    base = _wid() * P
    pltpu.sync_copy(x_hbm.at[pl.ds(base, P)], xv)
    pltpu.sync_copy(y_hbm.at[pl.ds(base, P)], yv)
    pltpu.sync_copy(z_hbm.at[pl.ds(base, P)], zv)

    def body(i, _):
        sl = pl.ds(i * L, L)
        xs, ys, zs = xv[sl], yv[sl], zv[sl]
        xi = jnp.clip((-ys / 0.2).astype(jnp.int32) + 200, 0, W - 1)
        yi = jnp.clip((-xs / 0.2).astype(jnp.int32) + 352, 0, H - 1)
        zb = (jnp.where(zs >= ZB[1], 1, 0) + jnp.where(zs >= ZB[2], 1, 0)
              + jnp.where(zs >= ZB[3], 1, 0) + jnp.where(zs >= ZB[4], 1, 0))
        valid = (zs >= ZB[0]) & (zs < ZB[5])
        cell = zb * (W * H) + (W - 1 - xi) * H + (H - 1 - yi)
        cv[sl] = jnp.where(valid, cell, SENT)
        return 0

    lax.fori_loop(0, P // L, body, 0)
    pltpu.sync_copy(cv, cells_hbm.at[pl.ds(base, P)])


CH = 8192


ZB_N = 16000
ZW = C * NCELLS // NW
ZREP = ZW // ZB_N


@functools.partial(
    pl.kernel, mesh=_mesh,
    out_type=jax.ShapeDtypeStruct((NCELLS,), jnp.int32),
    scratch_types=[
        pltpu.VMEM((CW,), jnp.int32),
        pltpu.VMEM((CH,), jnp.int32),
        pltpu.VMEM((ZB_N,), jnp.float32),
    ],
    compiler_params=pltpu.CompilerParams(needs_layout_passes=False),
)
def _winner_k(cells_hbm, out_fref, win_hbm, win_v, cb, zbuf):
    cbase = _wid() * CW
    zbase = _wid() * ZW

    def zinit(j, _):
        zbuf[pl.ds(j * L, L)] = jnp.zeros((L,), jnp.float32)
        return 0

    lax.fori_loop(0, ZB_N // L, zinit, 0)

    def zfill(k, _):
        pltpu.sync_copy(zbuf, out_fref.at[pl.ds(zbase + k * ZB_N, ZB_N)])
        return 0

    lax.fori_loop(0, ZREP, zfill, 0)

    def init(j, _):
        win_v[pl.ds(j * L, L)] = jnp.full((L,), -1, jnp.int32)
        return 0

    lax.fori_loop(0, CW // L, init, 0)

    def chunk(ch, _):
        pltpu.sync_copy(cells_hbm.at[pl.ds(ch * CH, CH)], cb)

        def body(j, _):
            cvv = cb[pl.ds(j * L, L)]
            n = ch * CH + j * L + lax.iota(jnp.int32, L)
            rel = cvv - cbase
            msk = (rel >= 0) & (rel < CW)
            relc = jnp.clip(rel, 0, CW - 1)
            plsc.store_scatter(win_v, [relc], n, mask=msk)
            return 0

        lax.fori_loop(0, CH // L, body, 0)
        return 0

    lax.fori_loop(0, N // CH, chunk, 0)
    pltpu.sync_copy(win_v, win_hbm.at[pl.ds(cbase, CW)])


CHK = 128
NCHMAX = P // CHK
NTMAX = CHK // L


@functools.partial(
    pl.kernel, mesh=_mesh,
    out_type=(),
    scratch_types=[
        pltpu.VMEM((P,), jnp.int32),
        pltpu.VMEM((P,), jnp.int32),
        pltpu.VMEM((P,), jnp.int32),
        pltpu.VMEM((P,), jnp.int32),
        pltpu.VMEM((NCHMAX, CHK), jnp.int32),
        pltpu.VMEM((NCHMAX, CHK), jnp.float32),
        pltpu.VMEM((NTMAX, L), jnp.int32),
        pltpu.VMEM((NTMAX, L), jnp.float32),
        pltpu.VMEM((P,), jnp.float32),
        pltpu.VMEM_SHARED((NCELLS,), jnp.int32),
        pltpu.SemaphoreType.DMA,
        pltpu.SemaphoreType.DMA,
    ],
    compiler_params=pltpu.CompilerParams(needs_layout_passes=False),
)
def _expand_k(cells_hbm, win_hbm, vals_hbm, out_ref,
              cell_v, gwin_v, off_v, ocell_v, idx2d, gv2d, idxt, gvt, vals_t,
              win_sh, gsem, ssem):
    sid = lax.axis_index("s")
    wid = _wid()
    pbase = wid * P

    @pl.when(sid == 0)
    def _():
        pltpu.sync_copy(win_hbm, win_sh)

    pltpu.sync_copy(cells_hbm.at[pl.ds(pbase, P)], cell_v)

    def clampc(j, _):
        sl = pl.ds(j * L, L)
        off_v[sl] = jnp.minimum(cell_v[sl], NCELLS - 1)
        return 0

    lax.fori_loop(0, P // L, clampc, 0)
    plsc.subcore_barrier()
    pltpu.async_copy(win_sh.at[off_v], gwin_v, gsem).wait()

    def compact(j, cnt):
        sl = pl.ds(j * L, L)
        lane = lax.iota(jnp.int32, L)
        m = gwin_v[sl] == pbase + j * L + lane
        plsc.store_compressed(off_v.at[pl.ds(cnt, L)], j * L + lane, mask=m)
        plsc.store_compressed(ocell_v.at[pl.ds(cnt, L)], cell_v[sl], mask=m)
        return cnt + jnp.max(plsc.all_reduce_population_count(m))

    cnt = lax.fori_loop(0, P // L, compact, jnp.int32(0))
    nfull = cnt >> 7
    nv16 = (cnt + L - 1) >> 4

    first_off = off_v[pl.ds(0, L)][0]
    first_cell = ocell_v[pl.ds(0, L)][0]

    def pad(k, _):
        sl = pl.ds(k * L, L)
        pos = k * L + lax.iota(jnp.int32, L)
        pm = pos >= cnt
        ocell_v[sl] = jnp.where(pm, first_cell, ocell_v[sl])
        off_v[sl] = jnp.where(pm, first_off, off_v[sl])
        return 0

    lax.fori_loop(cnt >> 4, nv16, pad, 0)

    def chan(c, _):
        pltpu.sync_copy(
            vals_hbm.at[pl.ds(pl.multiple_of(c * N + pbase, 8), P)], vals_t)

        def chunk(j, _):
            for u in range(CHK // L):
                sl = pl.ds(j * CHK + u * L, L)
                usl = pl.ds(u * L, L)
                g = plsc.load_gather(vals_t, [off_v[sl]])
                gv2d[j, usl] = g
                idx2d[j, usl] = ocell_v[sl] + c * NCELLS
            pltpu.async_copy(gv2d.at[j], out_ref.at[idx2d.at[j]], ssem)
            return 0

        lax.fori_loop(0, nfull, chunk, 0)

        def tail(v, _):
            t = v - (nfull << 3)
            sl = pl.ds(v * L, L)
            g = plsc.load_gather(vals_t, [off_v[sl]])
            gvt[t, :] = g
            idxt[t, :] = ocell_v[sl] + c * NCELLS
            pltpu.async_copy(gvt.at[t], out_ref.at[idxt.at[t]], ssem)
            return 0

        lax.fori_loop(nfull << 3, nv16, tail, 0)

        def drainc(j, _):
            pltpu.make_async_copy(gv2d.at[j], out_ref.at[idx2d.at[j]],
                                  ssem).wait()
            return 0

        lax.fori_loop(0, nfull, drainc, 0)

        def draint(v, _):
            t = v - (nfull << 3)
            pltpu.make_async_copy(gvt.at[t], out_ref.at[idxt.at[t]],
                                  ssem).wait()
            return 0

        lax.fori_loop(nfull << 3, nv16, draint, 0)
        return 0

    lax.fori_loop(0, C, chan, 0)


def kernel(range_res, rangemap_xyz):
    x = rangemap_xyz[0, 0].reshape(N)
    y = rangemap_xyz[0, 1].reshape(N)
    z = rangemap_xyz[0, 2].reshape(N)
    vals = range_res[0].reshape(C * N)
    cells = _cells_k(x, y, z)
    out_ref = pl.empty_ref_like(pltpu.HBM((C * NCELLS,), jnp.float32))
    win = _winner_k(cells, out_ref)
    _expand_k(cells, win, vals, out_ref)
    return out_ref[...].reshape(1, C, D, W, H)


# speedup vs baseline: 4.5637x; 4.5637x over previous
import functools

import jax
import jax.numpy as jnp
from jax import lax
from jax.experimental import pallas as pl
from jax.experimental.pallas import tpu as pltpu
from jax.experimental.pallas import tpu_sc as plsc

NC, NS, L = 2, 16, 16
NW = NC * NS
N = 131072
C = 64
D, W, H = 5, 400, 352
CP = 128
NCELLS = D * W * H
P = N // NW
CW = NCELLS // NW
SENT = 1 << 30

ZB = (-3.0, -2.2, -1.4000000000000004, -0.5999999999999996,
      0.20000000000000018, 1.0000000000000002)

_mesh = plsc.VectorSubcoreMesh(core_axis_name="c", subcore_axis_name="s")


def _wid():
    return lax.axis_index("s") * NC + lax.axis_index("c")


@functools.partial(
    pl.kernel, mesh=_mesh,
    out_type=jax.ShapeDtypeStruct((N,), jnp.int32),
    scratch_types=[
        pltpu.VMEM((P,), jnp.float32),
        pltpu.VMEM((P,), jnp.float32),
        pltpu.VMEM((P,), jnp.float32),
        pltpu.VMEM((P,), jnp.int32),
    ],
)
def _cells_k(x_hbm, y_hbm, z_hbm, cells_hbm, xv, yv, zv, cv):
    base = _wid() * P
    pltpu.sync_copy(x_hbm.at[pl.ds(base, P)], xv)
    pltpu.sync_copy(y_hbm.at[pl.ds(base, P)], yv)
    pltpu.sync_copy(z_hbm.at[pl.ds(base, P)], zv)

    def body(i, _):
        sl = pl.ds(i * L, L)
        xs, ys, zs = xv[sl], yv[sl], zv[sl]
        xi = jnp.clip((-ys / 0.2).astype(jnp.int32) + 200, 0, W - 1)
        yi = jnp.clip((-xs / 0.2).astype(jnp.int32) + 352, 0, H - 1)
        zb = (jnp.where(zs >= ZB[1], 1, 0) + jnp.where(zs >= ZB[2], 1, 0)
              + jnp.where(zs >= ZB[3], 1, 0) + jnp.where(zs >= ZB[4], 1, 0))
        valid = (zs >= ZB[0]) & (zs < ZB[5])
        cell = zb * (W * H) + (W - 1 - xi) * H + (H - 1 - yi)
        cv[sl] = jnp.where(valid, cell, SENT)
        return 0

    lax.fori_loop(0, P // L, body, 0)
    pltpu.sync_copy(cv, cells_hbm.at[pl.ds(base, P)])


CH = 8192


@functools.partial(
    pl.kernel, mesh=_mesh,
    out_type=jax.ShapeDtypeStruct((NCELLS,), jnp.int32),
    scratch_types=[
        pltpu.VMEM((CW,), jnp.int32),
        pltpu.VMEM((CH,), jnp.int32),
    ],
    compiler_params=pltpu.CompilerParams(needs_layout_passes=False),
)
def _winner_k(cells_hbm, win_hbm, win_v, cb):
    cbase = _wid() * CW

    def init(j, _):
        win_v[pl.ds(j * L, L)] = jnp.full((L,), -1, jnp.int32)
        return 0

    lax.fori_loop(0, CW // L, init, 0)

    def chunk(ch, _):
        pltpu.sync_copy(cells_hbm.at[pl.ds(ch * CH, CH)], cb)

        def body(j, _):
            cvv = cb[pl.ds(j * L, L)]
            n = ch * CH + j * L + lax.iota(jnp.int32, L)
            rel = cvv - cbase
            msk = (rel >= 0) & (rel < CW)
            relc = jnp.clip(rel, 0, CW - 1)
            plsc.store_scatter(win_v, [relc], n, mask=msk)
            return 0

        lax.fori_loop(0, CH // L, body, 0)
        return 0

    lax.fori_loop(0, N // CH, chunk, 0)
    pltpu.sync_copy(win_v, win_hbm.at[pl.ds(cbase, CW)])


CHK = 128
NCHMAX = P // CHK
NTMAX = CHK // L


@functools.partial(
    pl.kernel, mesh=_mesh,
    out_type=jax.ShapeDtypeStruct((NCELLS, CP), jnp.float32),
    scratch_types=[
        pltpu.VMEM((P,), jnp.int32),
        pltpu.VMEM((P,), jnp.int32),
        pltpu.VMEM((P,), jnp.int32),
        pltpu.VMEM((P,), jnp.int32),
        pltpu.VMEM((NCHMAX, CHK), jnp.int32),
        pltpu.VMEM((NTMAX, L), jnp.int32),
        pltpu.VMEM((CHK, CP), jnp.float32),
        pltpu.VMEM((L, CP), jnp.float32),
        pltpu.VMEM_SHARED((NCELLS,), jnp.int32),
        pltpu.SemaphoreType.DMA,
        pltpu.SemaphoreType.DMA,
    ],
    compiler_params=pltpu.CompilerParams(needs_layout_passes=False),
)
def _expand_k(cells_hbm, win_hbm, valst_hbm, out_hbm,
              cell_v, gwin_v, gn_v, ocell_v, ocell2d, ocellt,
              rows_v, rowst_v, win_sh, gsem, ssem):
    sid = lax.axis_index("s")
    wid = _wid()
    pbase = wid * P

    @pl.when(sid == 0)
    def _():
        pltpu.sync_copy(win_hbm, win_sh)

    pltpu.sync_copy(cells_hbm.at[pl.ds(pbase, P)], cell_v)

    def clampc(j, _):
        sl = pl.ds(j * L, L)
        gn_v[sl] = jnp.minimum(cell_v[sl], NCELLS - 1)
        return 0

    lax.fori_loop(0, P // L, clampc, 0)
    plsc.subcore_barrier()
    pltpu.async_copy(win_sh.at[gn_v], gwin_v, gsem).wait()

    def compact(j, cnt):
        sl = pl.ds(j * L, L)
        lane = lax.iota(jnp.int32, L)
        n = pbase + j * L + lane
        m = gwin_v[sl] == n
        plsc.store_compressed(gn_v.at[pl.ds(cnt, L)], n, mask=m)
        plsc.store_compressed(ocell_v.at[pl.ds(cnt, L)], cell_v[sl], mask=m)
        return cnt + jnp.max(plsc.all_reduce_population_count(m))

    cnt = lax.fori_loop(0, P // L, compact, jnp.int32(0))
    nfull = cnt >> 7
    nv16 = (cnt + L - 1) >> 4

    first_n = gn_v[pl.ds(0, L)][0]
    first_cell = ocell_v[pl.ds(0, L)][0]

    def pad(k, _):
        sl = pl.ds(k * L, L)
        pos = k * L + lax.iota(jnp.int32, L)
        pm = pos >= cnt
        ocell_v[sl] = jnp.where(pm, first_cell, ocell_v[sl])
        gn_v[sl] = jnp.where(pm, first_n, gn_v[sl])
        return 0

    lax.fori_loop(cnt >> 4, nv16, pad, 0)

    def repack(v, _):
        sl = pl.ds(v * L, L)
        j = v >> 3
        u = v - ((v >> 3) << 3)

        @pl.when(v < (nfull << 3))
        def _():
            ocell2d[j, pl.ds(u * L, L)] = ocell_v[sl]

        @pl.when(v >= (nfull << 3))
        def _():
            ocellt[v - (nfull << 3), :] = ocell_v[sl]

        return 0

    lax.fori_loop(0, nv16, repack, 0)

    def chunk(j, _):
        pltpu.async_copy(
            valst_hbm.at[gn_v.at[pl.ds(j * CHK, CHK)]], rows_v, gsem).wait()
        pltpu.async_copy(rows_v, out_hbm.at[ocell2d.at[j]], ssem).wait()
        return 0

    lax.fori_loop(0, nfull, chunk, 0)

    def tail(v, _):
        t = v - (nfull << 3)
        pltpu.async_copy(
            valst_hbm.at[gn_v.at[pl.ds(v * L, L)]], rowst_v, gsem).wait()
        pltpu.async_copy(rowst_v, out_hbm.at[ocellt.at[t]], ssem).wait()
        return 0

    lax.fori_loop(nfull << 3, nv16, tail, 0)


TBLK = 512


def _valst_body(v_ref, o_ref):
    t = jnp.transpose(v_ref[...], (1, 0))
    o_ref[...] = jnp.concatenate(
        [t, jnp.zeros((TBLK, CP - C), jnp.float32)], axis=1)


def _valst_k(vals2d):
    return pl.pallas_call(
        _valst_body,
        out_shape=jax.ShapeDtypeStruct((N, CP), jnp.float32),
        grid=(N // TBLK,),
        in_specs=[pl.BlockSpec((C, TBLK), lambda i: (0, i))],
        out_specs=pl.BlockSpec((TBLK, CP), lambda i: (i, 0)),
    )(vals2d)


def _final_body(t_ref, w_ref, o_ref):
    wb = w_ref[0, 0, :]
    t = jnp.transpose(t_ref[...], (1, 0))[:C, :]
    o_ref[...] = jnp.where((wb >= 0)[None, :], t, 0.0)


def _final_k(out2d, win3d):
    return pl.pallas_call(
        _final_body,
        out_shape=jax.ShapeDtypeStruct((C, NCELLS), jnp.float32),
        grid=(NCELLS // TBLK,),
        in_specs=[
            pl.BlockSpec((TBLK, CP), lambda i: (i, 0)),
            pl.BlockSpec((1, 1, TBLK), lambda i: (i, 0, 0)),
        ],
        out_specs=pl.BlockSpec((C, TBLK), lambda i: (0, i)),
    )(out2d, win3d)


def kernel(range_res, rangemap_xyz):
    x = rangemap_xyz[0, 0].reshape(N)
    y = rangemap_xyz[0, 1].reshape(N)
    z = rangemap_xyz[0, 2].reshape(N)
    vals2d = range_res[0].reshape(C, N)
    cells = _cells_k(x, y, z)
    win = _winner_k(cells)
    valst = _valst_k(vals2d)
    out2d = _expand_k(cells, win, valst)
    out = _final_k(out2d, win.reshape(NCELLS // TBLK, 1, TBLK))
    return out.reshape(1, C, D, W, H)


# speedup vs baseline: 8.1054x; 1.7760x over previous
import functools

import jax
import jax.numpy as jnp
from jax import lax
from jax.experimental import pallas as pl
from jax.experimental.pallas import tpu as pltpu
from jax.experimental.pallas import tpu_sc as plsc

NC, NS, L = 2, 16, 16
NW = NC * NS
N = 131072
C = 64
D, W, H = 5, 400, 352
CP = 128
NCELLS = D * W * H
P = N // NW
CW = NCELLS // NW
SENT = 1 << 30

ZB = (-3.0, -2.2, -1.4000000000000004, -0.5999999999999996,
      0.20000000000000018, 1.0000000000000002)

_mesh = plsc.VectorSubcoreMesh(core_axis_name="c", subcore_axis_name="s")


def _wid():
    return lax.axis_index("s") * NC + lax.axis_index("c")


@functools.partial(
    pl.kernel, mesh=_mesh,
    out_type=jax.ShapeDtypeStruct((N,), jnp.int32),
    scratch_types=[
        pltpu.VMEM((P,), jnp.float32),
        pltpu.VMEM((P,), jnp.float32),
        pltpu.VMEM((P,), jnp.float32),
        pltpu.VMEM((P,), jnp.int32),
    ],
)
def _cells_k(x_hbm, y_hbm, z_hbm, cells_hbm, xv, yv, zv, cv):
    base = _wid() * P
    pltpu.sync_copy(x_hbm.at[pl.ds(base, P)], xv)
    pltpu.sync_copy(y_hbm.at[pl.ds(base, P)], yv)
    pltpu.sync_copy(z_hbm.at[pl.ds(base, P)], zv)

    def body(i, _):
        sl = pl.ds(i * L, L)
        xs, ys, zs = xv[sl], yv[sl], zv[sl]
        xi = jnp.clip((-ys / 0.2).astype(jnp.int32) + 200, 0, W - 1)
        yi = jnp.clip((-xs / 0.2).astype(jnp.int32) + 352, 0, H - 1)
        zb = (jnp.where(zs >= ZB[1], 1, 0) + jnp.where(zs >= ZB[2], 1, 0)
              + jnp.where(zs >= ZB[3], 1, 0) + jnp.where(zs >= ZB[4], 1, 0))
        valid = (zs >= ZB[0]) & (zs < ZB[5])
        cell = zb * (W * H) + (W - 1 - xi) * H + (H - 1 - yi)
        cv[sl] = jnp.where(valid, cell, SENT)
        return 0

    lax.fori_loop(0, P // L, body, 0)
    pltpu.sync_copy(cv, cells_hbm.at[pl.ds(base, P)])


CH = 8192


@functools.partial(
    pl.kernel, mesh=_mesh,
    out_type=jax.ShapeDtypeStruct((NCELLS,), jnp.int32),
    scratch_types=[
        pltpu.VMEM((CW,), jnp.int32),
        pltpu.VMEM((CH,), jnp.int32),
    ],
    compiler_params=pltpu.CompilerParams(needs_layout_passes=False),
)
def _winner_k(cells_hbm, win_hbm, win_v, cb):
    cbase = _wid() * CW

    def init(j, _):
        win_v[pl.ds(j * L, L)] = jnp.full((L,), -1, jnp.int32)
        return 0

    lax.fori_loop(0, CW // L, init, 0)

    def chunk(ch, _):
        pltpu.sync_copy(cells_hbm.at[pl.ds(ch * CH, CH)], cb)

        def body(j, _):
            cvv = cb[pl.ds(j * L, L)]
            n = ch * CH + j * L + lax.iota(jnp.int32, L)
            rel = cvv - cbase
            msk = (rel >= 0) & (rel < CW)
            relc = jnp.clip(rel, 0, CW - 1)
            plsc.store_scatter(win_v, [relc], n, mask=msk)
            return 0

        lax.fori_loop(0, CH // L, body, 0)
        return 0

    lax.fori_loop(0, N // CH, chunk, 0)
    pltpu.sync_copy(win_v, win_hbm.at[pl.ds(cbase, CW)])


CHK = 128
NCHMAX = P // CHK
NTMAX = CHK // L


@functools.partial(
    pl.kernel, mesh=_mesh,
    out_type=jax.ShapeDtypeStruct((NCELLS, CP), jnp.float32),
    scratch_types=[
        pltpu.VMEM((P,), jnp.int32),
        pltpu.VMEM((P,), jnp.int32),
        pltpu.VMEM((P,), jnp.int32),
        pltpu.VMEM((P,), jnp.int32),
        pltpu.VMEM((NCHMAX, CHK), jnp.int32),
        pltpu.VMEM((NTMAX, L), jnp.int32),
        pltpu.VMEM((CHK, CP), jnp.float32),
        pltpu.VMEM((L, CP), jnp.float32),
        pltpu.VMEM_SHARED((NCELLS,), jnp.int32),
        pltpu.SemaphoreType.DMA,
        pltpu.SemaphoreType.DMA,
    ],
    compiler_params=pltpu.CompilerParams(needs_layout_passes=False),
)
def _expand_k(cells_hbm, win_hbm, valst_hbm, out_hbm,
              cell_v, gwin_v, gn_v, ocell_v, ocell2d, ocellt,
              rows_v, rowst_v, win_sh, gsem, ssem):
    sid = lax.axis_index("s")
    wid = _wid()
    pbase = wid * P

    @pl.when(sid == 0)
    def _():
        pltpu.sync_copy(win_hbm, win_sh)

    pltpu.sync_copy(cells_hbm.at[pl.ds(pbase, P)], cell_v)

    def clampc(j, _):
        sl = pl.ds(j * L, L)
        gn_v[sl] = jnp.minimum(cell_v[sl], NCELLS - 1)
        return 0

    lax.fori_loop(0, P // L, clampc, 0)
    plsc.subcore_barrier()
    pltpu.async_copy(win_sh.at[gn_v], gwin_v, gsem).wait()

    def compact(j, cnt):
        sl = pl.ds(j * L, L)
        lane = lax.iota(jnp.int32, L)
        n = pbase + j * L + lane
        m = gwin_v[sl] == n
        plsc.store_compressed(gn_v.at[pl.ds(cnt, L)], n, mask=m)
        plsc.store_compressed(ocell_v.at[pl.ds(cnt, L)], cell_v[sl], mask=m)
        return cnt + jnp.max(plsc.all_reduce_population_count(m))

    cnt = lax.fori_loop(0, P // L, compact, jnp.int32(0))
    nfull = cnt >> 7
    nv16 = (cnt + L - 1) >> 4

    first_n = gn_v[pl.ds(0, L)][0]
    first_cell = ocell_v[pl.ds(0, L)][0]

    def pad(k, _):
        sl = pl.ds(k * L, L)
        pos = k * L + lax.iota(jnp.int32, L)
        pm = pos >= cnt
        ocell_v[sl] = jnp.where(pm, first_cell, ocell_v[sl])
        gn_v[sl] = jnp.where(pm, first_n, gn_v[sl])
        return 0

    lax.fori_loop(cnt >> 4, nv16, pad, 0)

    def repack(v, _):
        sl = pl.ds(v * L, L)
        j = v >> 3
        u = v - ((v >> 3) << 3)

        @pl.when(v < (nfull << 3))
        def _():
            ocell2d[j, pl.ds(u * L, L)] = ocell_v[sl]

        @pl.when(v >= (nfull << 3))
        def _():
            ocellt[v - (nfull << 3), :] = ocell_v[sl]

        return 0

    lax.fori_loop(0, nv16, repack, 0)

    def chunk(j, _):
        pltpu.async_copy(
            valst_hbm.at[gn_v.at[pl.ds(j * CHK, CHK)]], rows_v, gsem).wait()
        pltpu.async_copy(rows_v, out_hbm.at[ocell2d.at[j]], ssem).wait()
        return 0

    lax.fori_loop(0, nfull, chunk, 0)

    def tail(v, _):
        t = v - (nfull << 3)
        pltpu.async_copy(
            valst_hbm.at[gn_v.at[pl.ds(v * L, L)]], rowst_v, gsem).wait()
        pltpu.async_copy(rowst_v, out_hbm.at[ocellt.at[t]], ssem).wait()
        return 0

    lax.fori_loop(nfull << 3, nv16, tail, 0)


TBLK = 2048
FBLK = 2560


def _valst_body(v_ref, o_ref):
    t = jnp.transpose(v_ref[...], (1, 0))
    o_ref[...] = jnp.concatenate(
        [t, jnp.zeros((TBLK, CP - C), jnp.float32)], axis=1)


def _valst_k(vals2d):
    return pl.pallas_call(
        _valst_body,
        out_shape=jax.ShapeDtypeStruct((N, CP), jnp.float32),
        grid=(N // TBLK,),
        in_specs=[pl.BlockSpec((C, TBLK), lambda i: (0, i))],
        out_specs=pl.BlockSpec((TBLK, CP), lambda i: (i, 0)),
    )(vals2d)


def _final_body(t_ref, w_ref, o_ref):
    wb = w_ref[0, 0, :]
    t = jnp.transpose(t_ref[...], (1, 0))[:C, :]
    o_ref[...] = jnp.where((wb >= 0)[None, :], t, 0.0)


def _final_k(out2d, win3d):
    return pl.pallas_call(
        _final_body,
        out_shape=jax.ShapeDtypeStruct((C, NCELLS), jnp.float32),
        grid=(NCELLS // FBLK,),
        in_specs=[
            pl.BlockSpec((FBLK, CP), lambda i: (i, 0)),
            pl.BlockSpec((1, 1, FBLK), lambda i: (i, 0, 0)),
        ],
        out_specs=pl.BlockSpec((C, FBLK), lambda i: (0, i)),
    )(out2d, win3d)


def kernel(range_res, rangemap_xyz):
    x = rangemap_xyz[0, 0].reshape(N)
    y = rangemap_xyz[0, 1].reshape(N)
    z = rangemap_xyz[0, 2].reshape(N)
    vals2d = range_res[0].reshape(C, N)
    cells = _cells_k(x, y, z)
    win = _winner_k(cells)
    valst = _valst_k(vals2d)
    out2d = _expand_k(cells, win, valst)
    out = _final_k(out2d, win.reshape(NCELLS // FBLK, 1, FBLK))
    return out.reshape(1, C, D, W, H)


# speedup vs baseline: 9.1266x; 1.1260x over previous
import functools

import jax
import jax.numpy as jnp
from jax import lax
from jax.experimental import pallas as pl
from jax.experimental.pallas import tpu as pltpu
from jax.experimental.pallas import tpu_sc as plsc

NC, NS, L = 2, 16, 16
NW = NC * NS
N = 131072
C = 64
D, W, H = 5, 400, 352
CP = 128
NCELLS = D * W * H
P = N // NW
CW = NCELLS // NW
SENT = 1 << 30

ZB = (-3.0, -2.2, -1.4000000000000004, -0.5999999999999996,
      0.20000000000000018, 1.0000000000000002)

_mesh = plsc.VectorSubcoreMesh(core_axis_name="c", subcore_axis_name="s")


def _wid():
    return lax.axis_index("s") * NC + lax.axis_index("c")


@functools.partial(
    pl.kernel, mesh=_mesh,
    out_type=jax.ShapeDtypeStruct((N,), jnp.int32),
    scratch_types=[
        pltpu.VMEM((P,), jnp.float32),
        pltpu.VMEM((P,), jnp.float32),
        pltpu.VMEM((P,), jnp.float32),
        pltpu.VMEM((P,), jnp.int32),
    ],
)
def _cells_k(x_hbm, y_hbm, z_hbm, cells_hbm, xv, yv, zv, cv):
    base = _wid() * P
    pltpu.sync_copy(x_hbm.at[pl.ds(base, P)], xv)
    pltpu.sync_copy(y_hbm.at[pl.ds(base, P)], yv)
    pltpu.sync_copy(z_hbm.at[pl.ds(base, P)], zv)

    def body(i, _):
        sl = pl.ds(i * L, L)
        xs, ys, zs = xv[sl], yv[sl], zv[sl]
        xi = jnp.clip((-ys / 0.2).astype(jnp.int32) + 200, 0, W - 1)
        yi = jnp.clip((-xs / 0.2).astype(jnp.int32) + 352, 0, H - 1)
        zb = (jnp.where(zs >= ZB[1], 1, 0) + jnp.where(zs >= ZB[2], 1, 0)
              + jnp.where(zs >= ZB[3], 1, 0) + jnp.where(zs >= ZB[4], 1, 0))
        valid = (zs >= ZB[0]) & (zs < ZB[5])
        cell = zb * (W * H) + (W - 1 - xi) * H + (H - 1 - yi)
        cv[sl] = jnp.where(valid, cell, SENT)
        return 0

    lax.fori_loop(0, P // L, body, 0)
    pltpu.sync_copy(cv, cells_hbm.at[pl.ds(base, P)])


CH = 8192


@functools.partial(
    pl.kernel, mesh=_mesh,
    out_type=jax.ShapeDtypeStruct((NCELLS,), jnp.int32),
    scratch_types=[
        pltpu.VMEM((CW,), jnp.int32),
        pltpu.VMEM((CH,), jnp.int32),
    ],
    compiler_params=pltpu.CompilerParams(needs_layout_passes=False),
)
def _winner_k(cells_hbm, win_hbm, win_v, cb):
    cbase = _wid() * CW

    def init(j, _):
        win_v[pl.ds(j * L, L)] = jnp.full((L,), -1, jnp.int32)
        return 0

    lax.fori_loop(0, CW // L, init, 0)

    def chunk(ch, _):
        pltpu.sync_copy(cells_hbm.at[pl.ds(ch * CH, CH)], cb)

        def body(j, _):
            cvv = cb[pl.ds(j * L, L)]
            n = ch * CH + j * L + lax.iota(jnp.int32, L)
            rel = cvv - cbase
            msk = (rel >= 0) & (rel < CW)
            relc = jnp.clip(rel, 0, CW - 1)
            plsc.store_scatter(win_v, [relc], n, mask=msk)
            return 0

        lax.fori_loop(0, CH // L, body, 0)
        return 0

    lax.fori_loop(0, N // CH, chunk, 0)
    pltpu.sync_copy(win_v, win_hbm.at[pl.ds(cbase, CW)])


CHK = 128
NCHMAX = P // CHK
NTMAX = CHK // L


@functools.partial(
    pl.kernel, mesh=_mesh,
    out_type=jax.ShapeDtypeStruct((NCELLS, CP), jnp.float32),
    scratch_types=[
        pltpu.VMEM((P,), jnp.int32),
        pltpu.VMEM((P,), jnp.int32),
        pltpu.VMEM((P,), jnp.int32),
        pltpu.VMEM((P,), jnp.int32),
        pltpu.VMEM((NCHMAX, CHK), jnp.int32),
        pltpu.VMEM((NTMAX, L), jnp.int32),
        pltpu.VMEM((CHK, CP), jnp.float32),
        pltpu.VMEM((L, CP), jnp.float32),
        pltpu.VMEM_SHARED((NCELLS,), jnp.int32),
        pltpu.SemaphoreType.DMA,
        pltpu.SemaphoreType.DMA,
    ],
    compiler_params=pltpu.CompilerParams(needs_layout_passes=False),
)
def _expand_k(cells_hbm, win_hbm, valst_hbm, out_hbm,
              cell_v, gwin_v, gn_v, ocell_v, ocell2d, ocellt,
              rows_v, rowst_v, win_sh, gsem, ssem):
    sid = lax.axis_index("s")
    wid = _wid()
    pbase = wid * P

    @pl.when(sid == 0)
    def _():
        pltpu.sync_copy(win_hbm, win_sh)

    pltpu.sync_copy(cells_hbm.at[pl.ds(pbase, P)], cell_v)

    def clampc(j, _):
        sl = pl.ds(j * L, L)
        gn_v[sl] = jnp.minimum(cell_v[sl], NCELLS - 1)
        return 0

    lax.fori_loop(0, P // L, clampc, 0)
    plsc.subcore_barrier()
    pltpu.async_copy(win_sh.at[gn_v], gwin_v, gsem).wait()

    def compact(j, cnt):
        sl = pl.ds(j * L, L)
        lane = lax.iota(jnp.int32, L)
        n = pbase + j * L + lane
        m = gwin_v[sl] == n
        plsc.store_compressed(gn_v.at[pl.ds(cnt, L)], n, mask=m)
        plsc.store_compressed(ocell_v.at[pl.ds(cnt, L)], cell_v[sl], mask=m)
        return cnt + jnp.max(plsc.all_reduce_population_count(m))

    cnt = lax.fori_loop(0, P // L, compact, jnp.int32(0))
    nfull = cnt >> 7
    nv16 = (cnt + L - 1) >> 4

    first_n = gn_v[pl.ds(0, L)][0]
    first_cell = ocell_v[pl.ds(0, L)][0]

    def pad(k, _):
        sl = pl.ds(k * L, L)
        pos = k * L + lax.iota(jnp.int32, L)
        pm = pos >= cnt
        ocell_v[sl] = jnp.where(pm, first_cell, ocell_v[sl])
        gn_v[sl] = jnp.where(pm, first_n, gn_v[sl])
        return 0

    lax.fori_loop(cnt >> 4, nv16, pad, 0)

    def repack(v, _):
        sl = pl.ds(v * L, L)
        j = v >> 3
        u = v - ((v >> 3) << 3)

        @pl.when(v < (nfull << 3))
        def _():
            ocell2d[j, pl.ds(u * L, L)] = ocell_v[sl]

        @pl.when(v >= (nfull << 3))
        def _():
            ocellt[v - (nfull << 3), :] = ocell_v[sl]

        return 0

    lax.fori_loop(0, nv16, repack, 0)

    def chunk(j, _):
        pltpu.async_copy(
            valst_hbm.at[gn_v.at[pl.ds(j * CHK, CHK)]], rows_v, gsem).wait()
        pltpu.async_copy(rows_v, out_hbm.at[ocell2d.at[j]], ssem).wait()
        return 0

    lax.fori_loop(0, nfull, chunk, 0)

    def tail(v, _):
        t = v - (nfull << 3)
        pltpu.async_copy(
            valst_hbm.at[gn_v.at[pl.ds(v * L, L)]], rowst_v, gsem).wait()
        pltpu.async_copy(rowst_v, out_hbm.at[ocellt.at[t]], ssem).wait()
        return 0

    lax.fori_loop(nfull << 3, nv16, tail, 0)


TBLK = 4096
FBLK = 5632


def _valst_body(v_ref, o_ref):
    t = jnp.transpose(v_ref[...], (1, 0))
    o_ref[...] = jnp.concatenate(
        [t, jnp.zeros((TBLK, CP - C), jnp.float32)], axis=1)


def _valst_k(vals2d):
    return pl.pallas_call(
        _valst_body,
        out_shape=jax.ShapeDtypeStruct((N, CP), jnp.float32),
        grid=(N // TBLK,),
        in_specs=[pl.BlockSpec((C, TBLK), lambda i: (0, i))],
        out_specs=pl.BlockSpec((TBLK, CP), lambda i: (i, 0)),
    )(vals2d)


def _final_body(t_ref, w_ref, o_ref):
    wb = w_ref[0, 0, :]
    t = jnp.transpose(t_ref[...], (1, 0))[:C, :]
    o_ref[...] = jnp.where((wb >= 0)[None, :], t, 0.0)


def _final_k(out2d, win3d):
    return pl.pallas_call(
        _final_body,
        out_shape=jax.ShapeDtypeStruct((C, NCELLS), jnp.float32),
        grid=(NCELLS // FBLK,),
        in_specs=[
            pl.BlockSpec((FBLK, CP), lambda i: (i, 0)),
            pl.BlockSpec((1, 1, FBLK), lambda i: (i, 0, 0)),
        ],
        out_specs=pl.BlockSpec((C, FBLK), lambda i: (0, i)),
    )(out2d, win3d)


def kernel(range_res, rangemap_xyz):
    x = rangemap_xyz[0, 0].reshape(N)
    y = rangemap_xyz[0, 1].reshape(N)
    z = rangemap_xyz[0, 2].reshape(N)
    vals2d = range_res[0].reshape(C, N)
    cells = _cells_k(x, y, z)
    win = _winner_k(cells)
    valst = _valst_k(vals2d)
    out2d = _expand_k(cells, win, valst)
    out = _final_k(out2d, win.reshape(NCELLS // FBLK, 1, FBLK))
    return out.reshape(1, C, D, W, H)
